# Initial kernel scaffold; baseline (speedup 1.0000x reference)
import jax
import jax.numpy as jnp
from jax.experimental import pallas as pl


def kernel(x, edge_index, W1, b1, W2, b2, Wf, bf):
    raise NotImplementedError("write your pallas kernel here")


# speedup vs baseline: 6.0159x; 6.0159x over previous
import functools

import jax
import jax.numpy as jnp
from jax import lax
from jax.experimental import pallas as pl
from jax.experimental.pallas import tpu as pltpu
from jax.experimental.pallas import tpu_sc as plsc

N = 10000
NPAD = 10240
E = 160000
EPAD = 163840
CHUNK = 128
D = 512
H = 256
HH = 128
NCLS = 100
CPAD = 128
BR = 256
PAD_SRC = N
PAD_DST = N + 16

ROWS_PER_TILE = NPAD // 16
EDGES_PER_TILE = EPAD // 16
CHUNKS_PER_TILE = EDGES_PER_TILE // CHUNK

_mesh = plsc.VectorSubcoreMesh(core_axis_name="c", subcore_axis_name="s")


@functools.partial(
    pl.kernel,
    out_type=jax.ShapeDtypeStruct((2, NPAD), jnp.float32),
    mesh=_mesh,
    scratch_types=[
        pltpu.VMEM((CHUNK,), jnp.int32),
        pltpu.VMEM((CHUNK,), jnp.float32),
        pltpu.VMEM((ROWS_PER_TILE,), jnp.float32),
        pltpu.VMEM_SHARED((NPAD,), jnp.float32),
    ],
)
def _deg_kernel(dst_hbm, out_hbm, dst_idx, ones_v, zbuf, hist):
    c = lax.axis_index("c")
    s = lax.axis_index("s")
    for j in range(ROWS_PER_TILE // 16):
        zbuf[pl.ds(j * 16, 16)] = jnp.zeros((16,), jnp.float32)
    for j in range(CHUNK // 16):
        ones_v[pl.ds(j * 16, 16)] = jnp.ones((16,), jnp.float32)
    row0 = s * ROWS_PER_TILE
    pltpu.sync_copy(zbuf, hist.at[pl.ds(row0, ROWS_PER_TILE)])
    plsc.subcore_barrier()

    ebase = c * (EPAD // 2) + s * (EPAD // 32)
    def chunk_body(i, carry):
        b = ebase + i * CHUNK
        pltpu.sync_copy(dst_hbm.at[pl.ds(b, CHUNK)], dst_idx)
        pltpu.sync_copy(ones_v, hist.at[dst_idx], add=True)
        return carry
    lax.fori_loop(0, (EPAD // 32) // CHUNK, chunk_body, 0)
    plsc.subcore_barrier()
    pltpu.sync_copy(hist.at[pl.ds(row0, ROWS_PER_TILE)],
                    out_hbm.at[c].at[pl.ds(row0, ROWS_PER_TILE)])


@functools.partial(
    pl.kernel,
    out_type=jax.ShapeDtypeStruct((2, NPAD, HH), jnp.float32),
    mesh=_mesh,
    scratch_types=[
        pltpu.VMEM((CHUNK,), jnp.int32),
        pltpu.VMEM((CHUNK,), jnp.int32),
        pltpu.VMEM((CHUNK, HH), jnp.float32),
        pltpu.VMEM_SHARED((NPAD, HH), jnp.float32),
        pltpu.SemaphoreType.DMA,
    ],
)
def _agg_kernel(y_hbm, src_hbm, dst_hbm, out_hbm, src_idx, dst_idx, gbuf,
                acc, sem):
    c = lax.axis_index("c")
    s = lax.axis_index("s")
    row0 = s * ROWS_PER_TILE
    pltpu.sync_copy(y_hbm.at[c].at[pl.ds(row0, ROWS_PER_TILE)],
                    acc.at[pl.ds(row0, ROWS_PER_TILE)])
    plsc.subcore_barrier()

    ebase = s * EDGES_PER_TILE
    def chunk_body(i, carry):
        b = ebase + i * CHUNK
        pltpu.sync_copy(src_hbm.at[pl.ds(b, CHUNK)], src_idx)
        pltpu.sync_copy(dst_hbm.at[pl.ds(b, CHUNK)], dst_idx)
        pltpu.async_copy(y_hbm.at[c].at[src_idx], gbuf, sem).wait()
        pltpu.sync_copy(gbuf, acc.at[dst_idx], add=True)
        return carry
    lax.fori_loop(0, CHUNKS_PER_TILE, chunk_body, 0)
    plsc.subcore_barrier()
    pltpu.sync_copy(acc.at[pl.ds(row0, ROWS_PER_TILE)],
                    out_hbm.at[c].at[pl.ds(row0, ROWS_PER_TILE)])


def _dis_from_hist(hist_blk):
    deg = hist_blk[0, :] + hist_blk[1, :] + 1.0
    return lax.rsqrt(jnp.maximum(deg, 1e-12))


def _mm1_body(hist_ref, x_ref, w_ref, y_ref):
    dis = _dis_from_hist(hist_ref)
    y = jnp.dot(x_ref[...], w_ref[...],
                preferred_element_type=jnp.float32) * dis[:, None]
    y_ref[0] = y[:, :HH]
    y_ref[1] = y[:, HH:]


def _mm2_body(hist_ref, a_ref, b1_ref, w_ref, y_ref):
    dis = _dis_from_hist(hist_ref)
    h = jnp.concatenate([a_ref[0], a_ref[1]], axis=1) * dis[:, None]
    h = jnp.maximum(h + b1_ref[...], 0.0)
    y = jnp.dot(h, w_ref[...],
                preferred_element_type=jnp.float32) * dis[:, None]
    y_ref[0] = y[:, :HH]
    y_ref[1] = y[:, HH:]


def _mm3_body(hist_ref, a_ref, b2_ref, wf_ref, bf_ref, o_ref):
    dis = _dis_from_hist(hist_ref)
    h = jnp.concatenate([a_ref[0], a_ref[1]], axis=1) * dis[:, None]
    h = jnp.maximum(h + b2_ref[...], 0.0)
    o_ref[...] = jnp.dot(h, wf_ref[...],
                         preferred_element_type=jnp.float32) + bf_ref[...]


_GRID = (NPAD // BR,)
_hist_spec = pl.BlockSpec((2, BR), lambda i: (0, i))
_half_spec = pl.BlockSpec((2, BR, HH), lambda i: (0, i, 0))
_bias_spec = pl.BlockSpec((1, H), lambda i: (0, 0))

_mm1 = pl.pallas_call(
    _mm1_body,
    grid=_GRID,
    in_specs=[
        _hist_spec,
        pl.BlockSpec((BR, D), lambda i: (i, 0)),
        pl.BlockSpec((D, H), lambda i: (0, 0)),
    ],
    out_specs=_half_spec,
    out_shape=jax.ShapeDtypeStruct((2, NPAD, HH), jnp.float32),
)

_mm2 = pl.pallas_call(
    _mm2_body,
    grid=_GRID,
    in_specs=[
        _hist_spec,
        _half_spec,
        _bias_spec,
        pl.BlockSpec((H, H), lambda i: (0, 0)),
    ],
    out_specs=_half_spec,
    out_shape=jax.ShapeDtypeStruct((2, NPAD, HH), jnp.float32),
)

_mm3 = pl.pallas_call(
    _mm3_body,
    grid=_GRID,
    in_specs=[
        _hist_spec,
        _half_spec,
        _bias_spec,
        pl.BlockSpec((H, CPAD), lambda i: (0, 0)),
        pl.BlockSpec((1, CPAD), lambda i: (0, 0)),
    ],
    out_specs=pl.BlockSpec((BR, CPAD), lambda i: (i, 0)),
    out_shape=jax.ShapeDtypeStruct((NPAD, CPAD), jnp.float32),
)


def kernel(x, edge_index, W1, b1, W2, b2, Wf, bf):
    xpad = jnp.pad(x, ((0, NPAD - N), (0, 0)))
    src = jnp.concatenate(
        [edge_index[0], jnp.full((EPAD - E,), PAD_SRC, jnp.int32)])
    dst = jnp.concatenate(
        [edge_index[1], jnp.full((EPAD - E,), PAD_DST, jnp.int32)])

    hist = _deg_kernel(dst)
    y1 = _mm1(hist, xpad, W1)
    agg1 = _agg_kernel(y1, src, dst)
    y2 = _mm2(hist, agg1, b1.reshape(1, H), W2)
    agg2 = _agg_kernel(y2, src, dst)
    wfp = jnp.pad(Wf, ((0, 0), (0, CPAD - NCLS)))
    bfp = jnp.pad(bf, (0, CPAD - NCLS)).reshape(1, CPAD)
    out = _mm3(hist, agg2, b2.reshape(1, H), wfp, bfp)
    return out[:N, :NCLS]


# speedup vs baseline: 8.2876x; 1.3776x over previous
import functools

import jax
import jax.numpy as jnp
from jax import lax
from jax.experimental import pallas as pl
from jax.experimental.pallas import tpu as pltpu
from jax.experimental.pallas import tpu_sc as plsc

N = 10000
NPAD = 10240
E = 160000
EPAD = 163840
CHUNK = 128
K = 2
NCHUNK = EPAD // 16 // CHUNK
D = 512
H = 256
HH = 128
NCLS = 100
CPAD = 128
BR = 256
PAD_SRC = N
PAD_DST = N + 16

ROWS_PER_TILE = NPAD // 16
EDGES_PER_TILE = EPAD // 16
CHUNKS_PER_TILE = EDGES_PER_TILE // CHUNK

_mesh = plsc.VectorSubcoreMesh(core_axis_name="c", subcore_axis_name="s")


@functools.partial(
    pl.kernel,
    out_type=jax.ShapeDtypeStruct((2, NPAD), jnp.float32),
    mesh=_mesh,
    scratch_types=[
        pltpu.VMEM((CHUNK,), jnp.int32),
        pltpu.VMEM((CHUNK,), jnp.int32),
        pltpu.VMEM((CHUNK,), jnp.float32),
        pltpu.VMEM((ROWS_PER_TILE,), jnp.float32),
        pltpu.VMEM_SHARED((NPAD,), jnp.float32),
        pltpu.SemaphoreType.DMA,
    ],
)
def _deg_kernel(dst_hbm, out_hbm, da, db, ones_v, zbuf, hist, sem):
    c = lax.axis_index("c")
    s = lax.axis_index("s")
    for j in range(ROWS_PER_TILE // 16):
        zbuf[pl.ds(j * 16, 16)] = jnp.zeros((16,), jnp.float32)
    for j in range(CHUNK // 16):
        ones_v[pl.ds(j * 16, 16)] = jnp.ones((16,), jnp.float32)
    row0 = s * ROWS_PER_TILE
    pltpu.sync_copy(zbuf, hist.at[pl.ds(row0, ROWS_PER_TILE)])
    plsc.subcore_barrier()

    NH = NCHUNK // 2
    ebase = s * EDGES_PER_TILE + c * (EDGES_PER_TILE // 2)
    bufs = (da, db)
    pend = [pltpu.async_copy(dst_hbm.at[pl.ds(ebase + b * CHUNK, CHUNK)],
                             bufs[b], sem)
            for b in range(2)]
    for i in range(NH):
        b = i % 2
        pend[b].wait()
        pltpu.sync_copy(ones_v, hist.at[bufs[b]], add=True)
        if i + 2 < NH:
            pend[b] = pltpu.async_copy(
                dst_hbm.at[pl.ds(ebase + (i + 2) * CHUNK, CHUNK)], bufs[b],
                sem)
    plsc.subcore_barrier()
    pltpu.sync_copy(hist.at[pl.ds(row0, ROWS_PER_TILE)],
                    out_hbm.at[c].at[pl.ds(row0, ROWS_PER_TILE)])


@functools.partial(
    pl.kernel,
    out_type=jax.ShapeDtypeStruct((2, NPAD, HH), jnp.float32),
    mesh=_mesh,
    scratch_types=[
        pltpu.VMEM((EDGES_PER_TILE,), jnp.int32),
        pltpu.VMEM((CHUNK,), jnp.int32),
        pltpu.VMEM((CHUNK,), jnp.int32),
        pltpu.VMEM((K, CHUNK, HH), jnp.float32),
        pltpu.VMEM_SHARED((NPAD, HH), jnp.float32),
        pltpu.SemaphoreType.DMA,
        pltpu.SemaphoreType.DMA,
    ],
)
def _agg_kernel(y_hbm, src_hbm, dst_hbm, out_hbm, src_i, da, db, gbuf,
                acc, gsa, dsem):
    c = lax.axis_index("c")
    s = lax.axis_index("s")
    row0 = s * ROWS_PER_TILE
    ebase = s * EDGES_PER_TILE
    pltpu.sync_copy(y_hbm.at[c].at[pl.ds(row0, ROWS_PER_TILE)],
                    acc.at[pl.ds(row0, ROWS_PER_TILE)])
    pltpu.sync_copy(src_hbm.at[pl.ds(ebase, EDGES_PER_TILE)], src_i)
    plsc.subcore_barrier()

    yc = y_hbm.at[c]
    dbufs = (da, db)

    pending = [pltpu.async_copy(yc.at[src_i.at[pl.ds(b * CHUNK, CHUNK)]],
                                gbuf.at[b], gsa)
               for b in range(K)]
    dpend = [pltpu.async_copy(
                 dst_hbm.at[pl.ds(ebase + b * CHUNK, CHUNK)], dbufs[b],
                 dsem)
             for b in range(2)]
    for i in range(NCHUNK):
        b = i % K
        d = i % 2
        pending[b].wait()
        dpend[d].wait()
        pltpu.sync_copy(gbuf.at[b], acc.at[dbufs[d]], add=True)
        if i + 2 < NCHUNK:
            dpend[d] = pltpu.async_copy(
                dst_hbm.at[pl.ds(ebase + (i + 2) * CHUNK, CHUNK)],
                dbufs[d], dsem)
        if i + K < NCHUNK:
            pending[b] = pltpu.async_copy(
                yc.at[src_i.at[pl.ds((i + K) * CHUNK, CHUNK)]],
                gbuf.at[b], gsa)
    plsc.subcore_barrier()
    pltpu.sync_copy(acc.at[pl.ds(row0, ROWS_PER_TILE)],
                    out_hbm.at[c].at[pl.ds(row0, ROWS_PER_TILE)])


def _dis_from_hist(hist_blk):
    deg = hist_blk[0, :] + hist_blk[1, :] + 1.0
    return lax.rsqrt(jnp.maximum(deg, 1e-12))


def _mm1_body(hist_ref, x_ref, w_ref, y_ref):
    dis = _dis_from_hist(hist_ref)
    y = jnp.dot(x_ref[...], w_ref[...],
                preferred_element_type=jnp.float32) * dis[:, None]
    y_ref[0] = y[:, :HH]
    y_ref[1] = y[:, HH:]


def _mm2_body(hist_ref, a_ref, b1_ref, w_ref, y_ref):
    dis = _dis_from_hist(hist_ref)
    h = jnp.concatenate([a_ref[0], a_ref[1]], axis=1) * dis[:, None]
    h = jnp.maximum(h + b1_ref[...], 0.0)
    y = jnp.dot(h, w_ref[...],
                preferred_element_type=jnp.float32) * dis[:, None]
    y_ref[0] = y[:, :HH]
    y_ref[1] = y[:, HH:]


def _mm3_body(hist_ref, a_ref, b2_ref, wf_ref, bf_ref, o_ref):
    dis = _dis_from_hist(hist_ref)
    h = jnp.concatenate([a_ref[0], a_ref[1]], axis=1) * dis[:, None]
    h = jnp.maximum(h + b2_ref[...], 0.0)
    o_ref[...] = jnp.dot(h, wf_ref[...],
                         preferred_element_type=jnp.float32) + bf_ref[...]


_GRID = (NPAD // BR,)
_hist_spec = pl.BlockSpec((2, BR), lambda i: (0, i))
_half_spec = pl.BlockSpec((2, BR, HH), lambda i: (0, i, 0))
_bias_spec = pl.BlockSpec((1, H), lambda i: (0, 0))

_mm1 = pl.pallas_call(
    _mm1_body,
    grid=_GRID,
    in_specs=[
        _hist_spec,
        pl.BlockSpec((BR, D), lambda i: (i, 0)),
        pl.BlockSpec((D, H), lambda i: (0, 0)),
    ],
    out_specs=_half_spec,
    out_shape=jax.ShapeDtypeStruct((2, NPAD, HH), jnp.float32),
)

_mm2 = pl.pallas_call(
    _mm2_body,
    grid=_GRID,
    in_specs=[
        _hist_spec,
        _half_spec,
        _bias_spec,
        pl.BlockSpec((H, H), lambda i: (0, 0)),
    ],
    out_specs=_half_spec,
    out_shape=jax.ShapeDtypeStruct((2, NPAD, HH), jnp.float32),
)

_mm3 = pl.pallas_call(
    _mm3_body,
    grid=_GRID,
    in_specs=[
        _hist_spec,
        _half_spec,
        _bias_spec,
        pl.BlockSpec((H, CPAD), lambda i: (0, 0)),
        pl.BlockSpec((1, CPAD), lambda i: (0, 0)),
    ],
    out_specs=pl.BlockSpec((BR, CPAD), lambda i: (i, 0)),
    out_shape=jax.ShapeDtypeStruct((NPAD, CPAD), jnp.float32),
)


def kernel(x, edge_index, W1, b1, W2, b2, Wf, bf):
    xpad = jnp.pad(x, ((0, NPAD - N), (0, 0)))
    src = jnp.concatenate(
        [edge_index[0], jnp.full((EPAD - E,), PAD_SRC, jnp.int32)])
    dst = jnp.concatenate(
        [edge_index[1], jnp.full((EPAD - E,), PAD_DST, jnp.int32)])

    hist = _deg_kernel(dst)
    y1 = _mm1(hist, xpad, W1)
    agg1 = _agg_kernel(y1, src, dst)
    y2 = _mm2(hist, agg1, b1.reshape(1, H), W2)
    agg2 = _agg_kernel(y2, src, dst)
    wfp = jnp.pad(Wf, ((0, 0), (0, CPAD - NCLS)))
    bfp = jnp.pad(bf, (0, CPAD - NCLS)).reshape(1, CPAD)
    out = _mm3(hist, agg2, b2.reshape(1, H), wfp, bfp)
    return out[:N, :NCLS]
